# baseline (speedup 1.0000x reference)
import functools

import jax
import jax.numpy as jnp
from jax import lax
from jax.experimental import pallas as pl
from jax.experimental.pallas import tpu as pltpu
from jax.experimental.pallas import tpu_sc as plsc

B = 128
S = 2048
D = 1024
NW_USED = 16

_mesh = plsc.VectorSubcoreMesh(core_axis_name="c", subcore_axis_name="s")


@functools.partial(
    pl.kernel,
    mesh=_mesh,
    out_type=jax.ShapeDtypeStruct((2 * B, D), jnp.float32),
    scratch_types=[
        pltpu.VMEM((2, B), jnp.int32),
        pltpu.VMEM((16,), jnp.int32),
        pltpu.VMEM((16,), jnp.int32),
        pltpu.VMEM((16, D), jnp.float32),
        pltpu.SemaphoreType.DMA,
    ],
)
def _gather_rows(table_hbm, pos1_hbm, pos2_hbm, out_hbm,
                 pos_v, idx_v, oidx_v, rows_v, sem):
    nc = 2
    wid = lax.axis_index("s") * nc + lax.axis_index("c")

    @pl.when(wid < NW_USED)
    def _():
        par = wid >> 3
        g = wid & 7
        pltpu.sync_copy(pos1_hbm, pos_v.at[0])
        pltpu.sync_copy(pos2_hbm, pos_v.at[1])
        j = lax.iota(jnp.int32, 16)
        bat = g * 16 + j
        pvec = pos_v[par, pl.ds(g * 16, 16)]
        idx_v[...] = bat * S + pvec
        oidx_v[...] = (bat << 1) + par
        pltpu.async_copy(table_hbm.at[idx_v], rows_v, sem).wait()
        pltpu.async_copy(rows_v, out_hbm.at[oidx_v], sem).wait()


def kernel(base_encoding, pos1, pos2):
    table = base_encoding.reshape(B * S, D)
    out = _gather_rows(table, pos1.astype(jnp.int32), pos2.astype(jnp.int32))
    return out.reshape(B, 2 * D)


# speedup vs baseline: 1.0843x; 1.0843x over previous
import functools

import jax
import jax.numpy as jnp
from jax import lax
from jax.experimental import pallas as pl
from jax.experimental.pallas import tpu as pltpu
from jax.experimental.pallas import tpu_sc as plsc

B = 128
S = 2048
D = 1024

_mesh = plsc.VectorSubcoreMesh(core_axis_name="c", subcore_axis_name="s",
                               num_cores=1)


@functools.partial(
    pl.kernel,
    mesh=_mesh,
    out_type=jax.ShapeDtypeStruct((2 * B, D), jnp.float32),
    scratch_types=[
        pltpu.VMEM((16,), jnp.int32),
        pltpu.VMEM((16,), jnp.int32),
        pltpu.VMEM((16,), jnp.int32),
        pltpu.VMEM((16,), jnp.int32),
        pltpu.VMEM((16, D), jnp.float32),
        pltpu.SemaphoreType.DMA,
    ],
)
def _gather_rows(table_hbm, pos1_hbm, pos2_hbm, out_hbm,
                 pos1_v, pos2_v, idx_v, oidx_v, rows_v, sem):
    wid = lax.axis_index("s")
    par = wid >> 3
    g = wid & 7

    c1 = pltpu.async_copy(pos1_hbm.at[pl.ds(g * 16, 16)], pos1_v, sem)
    c2 = pltpu.async_copy(pos2_hbm.at[pl.ds(g * 16, 16)], pos2_v, sem)
    c1.wait()
    c2.wait()

    j = lax.iota(jnp.int32, 16)
    bat = g * 16 + j
    idx_v[...] = bat * S + jnp.where(par == 0, pos1_v[...], pos2_v[...])
    oidx_v[...] = (bat << 1) + par
    pltpu.async_copy(table_hbm.at[idx_v], rows_v, sem).wait()
    pltpu.async_copy(rows_v, out_hbm.at[oidx_v], sem).wait()


def kernel(base_encoding, pos1, pos2):
    table = base_encoding.reshape(B * S, D)
    out = _gather_rows(table, pos1.astype(jnp.int32), pos2.astype(jnp.int32))
    return out.reshape(B, 2 * D)


# speedup vs baseline: 1.0946x; 1.0096x over previous
import functools

import jax
import jax.numpy as jnp
from jax import lax
from jax.experimental import pallas as pl
from jax.experimental.pallas import tpu as pltpu
from jax.experimental.pallas import tpu_sc as plsc

B = 128
S = 2048
D = 1024

_mesh = plsc.VectorSubcoreMesh(core_axis_name="c", subcore_axis_name="s",
                               num_cores=1)


@functools.partial(
    pl.kernel,
    mesh=_mesh,
    out_type=jax.ShapeDtypeStruct((2 * B, D), jnp.float32),
    scratch_types=[
        pltpu.VMEM((16,), jnp.int32),
        pltpu.VMEM((16,), jnp.int32),
        pltpu.VMEM((16,), jnp.int32),
        pltpu.VMEM((16,), jnp.int32),
        pltpu.VMEM((16, D), jnp.float32),
        pltpu.SemaphoreType.DMA,
    ],
)
def _gather_rows(table_hbm, pos1_hbm, pos2_hbm, out_hbm,
                 pos1_v, pos2_v, idx_v, oidx_v, rows_v, sem):
    wid = lax.axis_index("s")
    par = wid >> 3
    g = wid & 7

    c1 = pltpu.async_copy(pos1_hbm.at[pl.ds(g * 16, 16)], pos1_v, sem)
    c2 = pltpu.async_copy(pos2_hbm.at[pl.ds(g * 16, 16)], pos2_v, sem)
    c1.wait()
    c2.wait()

    j = lax.iota(jnp.int32, 16)
    bat = g * 16 + j
    idx = bat * S + jnp.where(par == 0, pos1_v[...], pos2_v[...])
    oidx = (bat << 1) + par
    pltpu.async_copy(table_hbm.at[idx], rows_v, sem).wait()
    pltpu.async_copy(rows_v, out_hbm.at[oidx], sem).wait()


def kernel(base_encoding, pos1, pos2):
    table = base_encoding.reshape(B * S, D)
    out = _gather_rows(table, pos1.astype(jnp.int32), pos2.astype(jnp.int32))
    return out.reshape(B, 2 * D)
